# baseline (speedup 1.0000x reference)
import functools

import jax
import jax.numpy as jnp
from jax import lax
from jax.experimental import pallas as pl
from jax.experimental.pallas import tpu as pltpu
from jax.experimental.pallas import tpu_sc as plsc

B = 4096
D = 768
K = 8192
PLEN = 20
HALF = (PLEN // 2) * D

BM = 2048
BK = 1024
NB = B // BM
NK = K // BK
EPS = 1e-12


def _topk_body(xq_ref, key_ref, outk_ref, outv_ref, best_ref, bidx_ref):
    kb = pl.program_id(1)
    xq = xq_ref[...]
    qn = xq / jnp.maximum(jnp.sqrt(jnp.sum(xq * xq, axis=1, keepdims=True)), EPS)
    kv = key_ref[...]
    kn = kv / jnp.maximum(jnp.sqrt(jnp.sum(kv * kv, axis=1, keepdims=True)), EPS)
    s = lax.dot_general(qn, kn, (((1,), (1,)), ((), ())),
                        preferred_element_type=jnp.float32)
    m = jnp.max(s, axis=1, keepdims=True)
    iota = lax.broadcasted_iota(jnp.int32, s.shape, 1)
    a = jnp.min(jnp.where(s == m, iota, K), axis=1, keepdims=True) + kb * BK

    @pl.when(kb == 0)
    def _():
        best_ref[...] = m
        bidx_ref[...] = a

    @pl.when(kb > 0)
    def _():
        prev_best = best_ref[...]
        prev_idx = bidx_ref[...]
        upd = m > prev_best
        best_ref[...] = jnp.where(upd, m, prev_best)
        bidx_ref[...] = jnp.where(upd, a, prev_idx)

    @pl.when(kb == NK - 1)
    def _():
        bi = bidx_ref[...]
        outk_ref[...] = bi * 2
        outv_ref[...] = bi * 2 + 1


def _top1_indices(xq, key):
    return pl.pallas_call(
        _topk_body,
        grid=(NB, NK),
        in_specs=[
            pl.BlockSpec((BM, D), lambda b, k: (b, 0)),
            pl.BlockSpec((BK, D), lambda b, k: (k, 0)),
        ],
        out_specs=[
            pl.BlockSpec((BM, 1), lambda b, k: (b, 0)),
            pl.BlockSpec((BM, 1), lambda b, k: (b, 0)),
        ],
        out_shape=[
            jax.ShapeDtypeStruct((B, 1), jnp.int32),
            jax.ShapeDtypeStruct((B, 1), jnp.int32),
        ],
        scratch_shapes=[
            pltpu.VMEM((BM, 1), jnp.float32),
            pltpu.VMEM((BM, 1), jnp.int32),
        ],
    )(xq, key)


NC = 2
NS = 16
NW = NC * NS
BPW = B // NW
CH = 8
NCHUNK = BPW // CH


@functools.cache
def _make_sc_gather():
    @functools.partial(
        pl.kernel,
        out_type=(
            jax.ShapeDtypeStruct((B, HALF), jnp.float32),
            jax.ShapeDtypeStruct((B, HALF), jnp.float32),
        ),
        mesh=plsc.VectorSubcoreMesh(core_axis_name="c", subcore_axis_name="s"),
        scratch_types=[
            pltpu.VMEM((BPW,), jnp.int32),
            pltpu.VMEM((BPW,), jnp.int32),
            pltpu.VMEM((CH, HALF), jnp.float32),
            pltpu.SemaphoreType.DMA,
        ],
    )
    def _sc_gather(p0_hbm, idxk_hbm, idxv_hbm, pk_hbm, pv_hbm,
                   idxk_v, idxv_v, buf, sem):
        wid = lax.axis_index("s") * NC + lax.axis_index("c")
        base = wid * BPW
        pltpu.sync_copy(idxk_hbm.at[pl.ds(base, BPW)], idxk_v)
        pltpu.sync_copy(idxv_hbm.at[pl.ds(base, BPW)], idxv_v)

        def body(j, carry):
            off = j * CH
            pltpu.async_copy(p0_hbm.at[idxk_v.at[pl.ds(off, CH)]], buf, sem).wait()
            pltpu.sync_copy(buf, pk_hbm.at[pl.ds(base + off, CH)])
            pltpu.async_copy(p0_hbm.at[idxv_v.at[pl.ds(off, CH)]], buf, sem).wait()
            pltpu.sync_copy(buf, pv_hbm.at[pl.ds(base + off, CH)])
            return carry

        lax.fori_loop(0, NCHUNK, body, 0)

    return _sc_gather


def kernel(l, x_block, x_query, key, p0):
    del l
    xq = x_query.reshape(B, D)
    idxk, idxv = _top1_indices(xq, key)
    p0f = p0.reshape(K * 2, HALF)
    pk, pv = _make_sc_gather()(p0f, idxk.reshape(B), idxv.reshape(B))
    return (pk.reshape(B, PLEN // 2, D), pv.reshape(B, PLEN // 2, D), x_block)
